# baseline (speedup 1.0000x reference)
import functools

import jax
import jax.numpy as jnp
from jax import lax
from jax.experimental import pallas as pl
from jax.experimental.pallas import tpu as pltpu
from jax.experimental.pallas import tpu_sc as plsc

_LANES = 16


def _sc_gather_pos(ar_flat, table):
    n_rows_out = 32
    width = table.shape[1]
    num_tiles = 4
    zero_row = table.shape[0] - 1

    mesh = plsc.VectorSubcoreMesh(core_axis_name="c", subcore_axis_name="s")

    @functools.partial(
        pl.kernel,
        mesh=mesh,
        compiler_params=pltpu.CompilerParams(needs_layout_passes=False),
        out_type=jax.ShapeDtypeStruct((n_rows_out, width), jnp.float32),
        scratch_types=[
            pltpu.VMEM((_LANES,), jnp.int32),
            pltpu.VMEM((1,), jnp.int32),
            pltpu.VMEM((1, width), jnp.float32),
            pltpu.SemaphoreType.DMA,
        ],
    )
    def sc_kernel(ar_hbm, table_hbm, out_hbm, ar_v, idx_v, row_v, sem):
        wid = lax.axis_index("s") * 2 + lax.axis_index("c")
        pltpu.sync_copy(ar_hbm, ar_v)
        lane = lax.iota(jnp.int32, _LANES)
        k = lane + (wid // _LANES) * _LANES
        b2 = (k // num_tiles) * 2
        w = plsc.load_gather(ar_v, [b2])
        h = plsc.load_gather(ar_v, [b2 + 1])
        h_safe = jnp.maximum(h, 1)
        t = k % num_tiles
        idx = jnp.where(
            t < w * h,
            (t // h_safe) * num_tiles + (t % h_safe),
            zero_row,
        )
        plsc.store_scatter(idx_v, [lane * 0], idx, mask=lane == (wid % _LANES))
        pltpu.async_copy(table_hbm.at[idx_v], row_v, sem).wait()
        pltpu.sync_copy(row_v, out_hbm.at[pl.ds(wid, 1)])

    return sc_kernel(ar_flat, table)


def _tc_add(x, pos):
    B, T, N, D = x.shape

    NB = 512
    nchunks = (N + NB - 1) // NB

    def body(x_ref, p_ref, o_ref):
        o_ref[0, 0] = x_ref[0, 0] + p_ref[0]

    grid = (B * T, nchunks)
    return pl.pallas_call(
        body,
        grid=grid,
        in_specs=[
            pl.BlockSpec((1, 1, NB, D), lambda i, j: (i // T, i % T, j, 0)),
            pl.BlockSpec((1, 1, D), lambda i, j: (i, 0, 0)),
        ],
        out_specs=pl.BlockSpec(
            (1, 1, NB, D), lambda i, j: (i // T, i % T, j, 0)
        ),
        out_shape=jax.ShapeDtypeStruct(x.shape, x.dtype),
        compiler_params=pltpu.CompilerParams(
            dimension_semantics=("parallel", "parallel"),
        ),
    )(x, pos.reshape(B * T, 1, D))


def kernel(x, ar, embedding):
    B, T, N, D = x.shape
    nt = embedding.shape[0]
    ar_flat = ar.astype(jnp.int32).reshape(-1)
    table = jnp.concatenate(
        [embedding.reshape(nt * nt, D), jnp.zeros((1, D), embedding.dtype)],
        axis=0,
    )
    pos = _sc_gather_pos(ar_flat, table)
    return _tc_add(x, pos)


# speedup vs baseline: 1.0387x; 1.0387x over previous
import functools

import jax
import jax.numpy as jnp
from jax import lax
from jax.experimental import pallas as pl
from jax.experimental.pallas import tpu as pltpu
from jax.experimental.pallas import tpu_sc as plsc

_LANES = 16


def _sc_gather_pos(ar_flat, table):
    n_rows_out = 32
    width = table.shape[1]
    num_tiles = 4
    zero_row = table.shape[0] - 1

    mesh = plsc.VectorSubcoreMesh(core_axis_name="c", subcore_axis_name="s")

    @functools.partial(
        pl.kernel,
        mesh=mesh,
        compiler_params=pltpu.CompilerParams(needs_layout_passes=False),
        out_type=jax.ShapeDtypeStruct((n_rows_out, width), jnp.float32),
        scratch_types=[
            pltpu.VMEM((_LANES,), jnp.int32),
            pltpu.VMEM((1,), jnp.int32),
            pltpu.VMEM((1, width), jnp.float32),
            pltpu.SemaphoreType.DMA,
        ],
    )
    def sc_kernel(ar_hbm, table_hbm, out_hbm, ar_v, idx_v, row_v, sem):
        wid = lax.axis_index("s") * 2 + lax.axis_index("c")
        pltpu.sync_copy(ar_hbm, ar_v)
        lane = lax.iota(jnp.int32, _LANES)
        k = lane + (wid // _LANES) * _LANES
        b2 = (k // num_tiles) * 2
        w = plsc.load_gather(ar_v, [b2])
        h = plsc.load_gather(ar_v, [b2 + 1])
        h_safe = jnp.maximum(h, 1)
        t = k % num_tiles
        idx = jnp.where(
            t < w * h,
            (t // h_safe) * num_tiles + (t % h_safe),
            zero_row,
        )
        plsc.store_scatter(idx_v, [lane * 0], idx, mask=lane == (wid % _LANES))
        pltpu.async_copy(table_hbm.at[idx_v], row_v, sem).wait()
        pltpu.sync_copy(row_v, out_hbm.at[pl.ds(wid, 1)])

    return sc_kernel(ar_flat, table)


def _tc_add(x, pos):
    B, T, N, D = x.shape
    NB = 256
    CPS = N // NB
    TAIL = N - CPS * NB
    NCHUNK = B * T * CPS
    NBUF = 8
    NOUTER = NCHUNK // NBUF

    def body(x_hbm, pos_v, out_hbm, in_buf, out_buf, tail_in, tail_out,
             in_sems, out_sems, tail_in_sem, tail_out_sem):
        def in_copy(i, s):
            bt = i // CPS
            r = (i % CPS) * NB
            return pltpu.make_async_copy(
                x_hbm.at[bt // T, bt % T, pl.ds(r, NB), :],
                in_buf.at[s],
                in_sems.at[s],
            )

        def out_copy(i, s):
            bt = i // CPS
            r = (i % CPS) * NB
            return pltpu.make_async_copy(
                out_buf.at[s],
                out_hbm.at[bt // T, bt % T, pl.ds(r, NB), :],
                out_sems.at[s],
            )

        def tail_in_copy():
            return pltpu.make_async_copy(
                x_hbm.at[:, :, pl.ds(CPS * NB, TAIL), :],
                tail_in,
                tail_in_sem,
            )

        tail_in_copy().start()
        for s in range(NBUF):
            in_copy(s, s).start()

        def step(i, s):
            bt = i // CPS
            in_copy(i, s).wait()
            out_buf[s] = in_buf[s] + pos_v[bt // T, bt % T]
            out_copy(i, s).start()

        for s in range(NBUF):
            step(s, s)
            in_copy(s + NBUF, s).start()

        def outer(o, carry):
            for s in range(NBUF):
                i = o * NBUF + s
                out_copy(i - NBUF, s).wait()
                step(i, s)

                @pl.when(o < NOUTER - 1)
                def _():
                    in_copy(i + NBUF, s).start()

            return carry

        lax.fori_loop(1, NOUTER, outer, 0)

        tail_in_copy().wait()
        tail_out[...] = tail_in[...] + pos_v[...]
        tail_cp = pltpu.make_async_copy(
            tail_out, out_hbm.at[:, :, pl.ds(CPS * NB, TAIL), :], tail_out_sem
        )
        tail_cp.start()

        for s in range(NBUF):
            out_copy(NCHUNK - NBUF + s, s).wait()
        tail_cp.wait()

    return pl.pallas_call(
        body,
        in_specs=[
            pl.BlockSpec(memory_space=pl.ANY),
            pl.BlockSpec(memory_space=pltpu.MemorySpace.VMEM),
        ],
        out_specs=pl.BlockSpec(memory_space=pl.ANY),
        out_shape=jax.ShapeDtypeStruct(x.shape, x.dtype),
        scratch_shapes=[
            pltpu.VMEM((NBUF, NB, D), x.dtype),
            pltpu.VMEM((NBUF, NB, D), x.dtype),
            pltpu.VMEM((B, T, TAIL, D), x.dtype),
            pltpu.VMEM((B, T, TAIL, D), x.dtype),
            pltpu.SemaphoreType.DMA((NBUF,)),
            pltpu.SemaphoreType.DMA((NBUF,)),
            pltpu.SemaphoreType.DMA,
            pltpu.SemaphoreType.DMA,
        ],
        compiler_params=pltpu.CompilerParams(
            vmem_limit_bytes=100 * 1024 * 1024,
        ),
    )(x, pos.reshape(B, T, 1, D))


def kernel(x, ar, embedding):
    B, T, N, D = x.shape
    nt = embedding.shape[0]
    ar_flat = ar.astype(jnp.int32).reshape(-1)
    table = jnp.concatenate(
        [embedding.reshape(nt * nt, D), jnp.zeros((1, D), embedding.dtype)],
        axis=0,
    )
    pos = _sc_gather_pos(ar_flat, table)
    return _tc_add(x, pos)
